# baseline (speedup 1.0000x reference)
import jax
import jax.numpy as jnp
from jax.experimental import pallas as pl
from jax.experimental.pallas import tpu as pltpu

D_MODEL = 256
ROW_BLOCK = 1000


def _proj_kernel(ctx_ref, smp_ref, w_ref, b_ref, out_ref):
    w = w_ref[...]
    b = b_ref[...]
    out_ref[0] = jnp.dot(ctx_ref[...], w, preferred_element_type=jnp.float32) + b
    out_ref[1] = jnp.dot(smp_ref[...], w, preferred_element_type=jnp.float32) + b


def kernel(context, sample, W_proj, b_proj):
    n, d = context.shape
    blk = ROW_BLOCK if n % ROW_BLOCK == 0 else n
    nb = n // blk
    b2 = b_proj.reshape(1, d)
    out = pl.pallas_call(
        _proj_kernel,
        grid=(nb,),
        in_specs=[
            pl.BlockSpec((blk, d), lambda i: (i, 0)),
            pl.BlockSpec((blk, d), lambda i: (i, 0)),
            pl.BlockSpec((d, d), lambda i: (0, 0)),
            pl.BlockSpec((1, d), lambda i: (0, 0)),
        ],
        out_specs=pl.BlockSpec((2, blk, d), lambda i: (0, i, 0)),
        out_shape=jax.ShapeDtypeStruct((2, n, d), jnp.float32),
        compiler_params=pltpu.CompilerParams(
            dimension_semantics=("parallel",),
        ),
    )(context, sample, W_proj, b2)
    return out.reshape(2 * n, d)


# speedup vs baseline: 1.2187x; 1.2187x over previous
import jax
import jax.numpy as jnp
from jax.experimental import pallas as pl
from jax.experimental.pallas import tpu as pltpu

D_MODEL = 256
ROW_BLOCK = 5000


def _proj_kernel(ctx_ref, smp_ref, w_ref, b_ref, out_ref):
    w = w_ref[...]
    b = b_ref[...]
    out_ref[0] = jnp.dot(ctx_ref[...], w, preferred_element_type=jnp.float32) + b
    out_ref[1] = jnp.dot(smp_ref[...], w, preferred_element_type=jnp.float32) + b


def kernel(context, sample, W_proj, b_proj):
    n, d = context.shape
    blk = ROW_BLOCK if n % ROW_BLOCK == 0 else n
    nb = n // blk
    b2 = b_proj.reshape(1, d)
    out = pl.pallas_call(
        _proj_kernel,
        grid=(nb,),
        in_specs=[
            pl.BlockSpec((blk, d), lambda i: (i, 0)),
            pl.BlockSpec((blk, d), lambda i: (i, 0)),
            pl.BlockSpec((d, d), lambda i: (0, 0)),
            pl.BlockSpec((1, d), lambda i: (0, 0)),
        ],
        out_specs=pl.BlockSpec((2, blk, d), lambda i: (0, i, 0)),
        out_shape=jax.ShapeDtypeStruct((2, n, d), jnp.float32),
        compiler_params=pltpu.CompilerParams(
            dimension_semantics=("parallel",),
        ),
    )(context, sample, W_proj, b2)
    return out.reshape(2 * n, d)


# speedup vs baseline: 1.2544x; 1.0293x over previous
import jax
import jax.numpy as jnp
from jax.experimental import pallas as pl
from jax.experimental.pallas import tpu as pltpu

D_MODEL = 256
ROW_BLOCK = 6256


def _proj_kernel(ctx_ref, smp_ref, w_ref, b_ref, out_ref):
    w = w_ref[...]
    b = b_ref[...]
    out_ref[0] = jnp.dot(ctx_ref[...], w, preferred_element_type=jnp.float32) + b
    out_ref[1] = jnp.dot(smp_ref[...], w, preferred_element_type=jnp.float32) + b


def kernel(context, sample, W_proj, b_proj):
    n, d = context.shape
    blk = min(ROW_BLOCK, n)
    nb = pl.cdiv(n, blk)
    b2 = b_proj.reshape(1, d)
    out = pl.pallas_call(
        _proj_kernel,
        grid=(nb,),
        in_specs=[
            pl.BlockSpec((blk, d), lambda i: (i, 0)),
            pl.BlockSpec((blk, d), lambda i: (i, 0)),
            pl.BlockSpec((d, d), lambda i: (0, 0)),
            pl.BlockSpec((1, d), lambda i: (0, 0)),
        ],
        out_specs=pl.BlockSpec((2, blk, d), lambda i: (0, i, 0)),
        out_shape=jax.ShapeDtypeStruct((2, n, d), jnp.float32),
        compiler_params=pltpu.CompilerParams(
            dimension_semantics=("parallel",),
        ),
    )(context, sample, W_proj, b2)
    return out.reshape(2 * n, d)
